# Initial kernel scaffold; baseline (speedup 1.0000x reference)
import jax
import jax.numpy as jnp
from jax.experimental import pallas as pl


def kernel(pointcloud, params):
    raise NotImplementedError("write your pallas kernel here")


# speedup vs baseline: 41.9461x; 41.9461x over previous
import functools

import jax
import jax.numpy as jnp
from jax import lax
from jax.experimental import pallas as pl
from jax.experimental.pallas import tpu as pltpu
from jax.experimental.pallas import tpu_sc as plsc

_NC, _NS = 2, 16
_NW = _NC * _NS


def _pad16(c):
    return (c + 15) // 16 * 16


def _fps_body(x_ref, y_ref, z_ref, cx_ref, cy_ref, cz_ref, *, N, S):
    x = x_ref[...]
    y = y_ref[...]
    z = z_ref[...]
    B = x.shape[0]
    col = lax.broadcasted_iota(jnp.int32, (B, N), 1)
    row = lax.broadcasted_iota(jnp.int32, (B, N), 0)

    col_s = lax.broadcasted_iota(jnp.int32, (B, S), 1)
    row_s = lax.broadcasted_iota(jnp.int32, (B, S), 0)

    def body(i, state):
        dists, far, cxs, cys, czs = state
        sel = col_s == i
        cmask = col == far
        cx = jnp.sum(jnp.where(cmask, x, 0.0), axis=1, keepdims=True)
        cy = jnp.sum(jnp.where(cmask, y, 0.0), axis=1, keepdims=True)
        cz = jnp.sum(jnp.where(cmask, z, 0.0), axis=1, keepdims=True)
        cxs = jnp.where(sel, cx, cxs)
        cys = jnp.where(sel, cy, cys)
        czs = jnp.where(sel, cz, czs)
        d = ((x - cx) ** 2 + (y - cy) ** 2) + (z - cz) ** 2
        dists = jnp.minimum(dists, d)
        mx = jnp.max(dists, axis=1, keepdims=True)
        far = jnp.min(jnp.where(dists == mx, col, N), axis=1, keepdims=True)
        return dists, far, cxs, cys, czs

    zero2d = (row + col) * 0
    zs_f = ((row_s + col_s) * 0).astype(jnp.float32)
    init = (
        zero2d.astype(jnp.float32) + jnp.float32(1e10),
        jnp.min(zero2d, axis=1, keepdims=True),
        zs_f,
        zs_f,
        zs_f,
    )
    _, _, cxs, cys, czs = lax.fori_loop(0, S, body, init)
    cx_ref[...] = cxs
    cy_ref[...] = cys
    cz_ref[...] = czs


def _fps(x, y, z, S):
    B, N = x.shape
    out = (
        jax.ShapeDtypeStruct((B, S), jnp.float32),
        jax.ShapeDtypeStruct((B, S), jnp.float32),
        jax.ShapeDtypeStruct((B, S), jnp.float32),
    )
    return pl.pallas_call(
        functools.partial(_fps_body, N=N, S=S),
        out_shape=out,
    )(x, y, z)


def _bq_body(nxp_ref, xpt_ref, gidx_ref, *, N, K, r2, St):
    b = pl.program_id(0)
    nx = nxp_ref[0]
    xt = xpt_ref[0]
    dot = jnp.dot(nx, xt, preferred_element_type=jnp.float32)
    ns = (nx[:, 0:1] * nx[:, 0:1] + nx[:, 1:2] * nx[:, 1:2]) + nx[:, 2:3] * nx[:, 2:3]
    xs = (xt[0:1] * xt[0:1] + xt[1:2] * xt[1:2]) + xt[2:3] * xt[2:3]
    sq = ns + xs - 2.0 * dot
    col = lax.broadcasted_iota(jnp.int32, (St, N), 1)
    keys = jnp.where(sq <= r2, col, N)
    first = jnp.min(keys, axis=1, keepdims=True)
    col_k = lax.broadcasted_iota(jnp.int32, (St, K), 1)

    def body(k, state):
        keys, out = state
        m = jnp.min(keys, axis=1, keepdims=True)
        keys = jnp.where(keys == m, N, keys)
        val = jnp.where(m == N, first, m)
        out = jnp.where(col_k == k, val, out)
        return keys, out

    _, out = lax.fori_loop(0, K, body, (keys, jnp.zeros((St, K), jnp.int32)))
    gidx_ref[0] = out + b * N


def _ball_query(nxp, xpt, radius, K, St):
    B, S, _ = nxp.shape
    N = xpt.shape[2]
    r2 = float(radius * radius)
    return pl.pallas_call(
        functools.partial(_bq_body, N=N, K=K, r2=r2, St=St),
        grid=(B, S // St),
        in_specs=[
            pl.BlockSpec((1, St, 8), lambda b, s: (b, s, 0)),
            pl.BlockSpec((1, 8, N), lambda b, s: (b, 0, 0)),
        ],
        out_specs=pl.BlockSpec((1, St, K), lambda b, s: (b, s, 0)),
        out_shape=jax.ShapeDtypeStruct((B, S, K), jnp.int32),
    )(nxp, xpt)


def _sc_gather(table, idx, CH):
    R, D = table.shape
    M = idx.shape[0]
    per_w = M // _NW
    assert per_w * _NW == M and per_w % CH == 0 and D % 16 == 0
    mesh = plsc.VectorSubcoreMesh(core_axis_name="c", subcore_axis_name="s",
                                  num_cores=_NC, num_subcores=_NS)

    @functools.partial(
        pl.kernel,
        mesh=mesh,
        out_type=jax.ShapeDtypeStruct((M, D), jnp.float32),
        scratch_types=[
            pltpu.VMEM((CH,), jnp.int32),
            pltpu.VMEM((CH, D), jnp.float32),
            pltpu.SemaphoreType.DMA,
        ],
        compiler_params=pltpu.CompilerParams(use_tc_tiling_on_sc=False),
    )
    def k(table_hbm, idx_hbm, out_hbm, idx_v, rows_v, sem):
        wid = lax.axis_index("s") * _NC + lax.axis_index("c")
        base = wid * per_w

        def step(i, carry):
            off = base + i * CH
            pltpu.sync_copy(idx_hbm.at[pl.ds(off, CH)], idx_v)
            pltpu.async_copy(table_hbm.at[idx_v], rows_v, sem).wait()
            pltpu.sync_copy(rows_v, out_hbm.at[pl.ds(off, CH)])
            return carry

        lax.fori_loop(0, per_w // CH, step, 0)

    return k(table, idx)


def _mlp_body(g_ref, c_ref, w1_ref, b1_ref, wc_ref, w2_ref, b2_ref, w3_ref,
              b3_ref, o_ref, *, St, K):
    g = g_ref[...]
    c = c_ref[...]
    cterm = jnp.dot(c, wc_ref[...], preferred_element_type=jnp.float32)
    h = jnp.dot(g, w1_ref[...], preferred_element_type=jnp.float32) + b1_ref[...]
    h = h.reshape(St, K, -1) - cterm[:, None, :]
    h = jnp.maximum(h, 0.0).reshape(St * K, h.shape[-1])
    h = jnp.maximum(jnp.dot(h, w2_ref[...], preferred_element_type=jnp.float32)
                    + b2_ref[...], 0.0)
    h = jnp.maximum(jnp.dot(h, w3_ref[...], preferred_element_type=jnp.float32)
                    + b3_ref[...], 0.0)
    o_ref[...] = jnp.max(h.reshape(St, K, h.shape[-1]), axis=1)


def _group_mlp(G, cent, layers, K, St):
    M, Dp = G.shape
    SB = M // K
    (w1, b1), (w2, b2), (w3, b3) = layers
    cin, c1 = w1.shape
    c2 = w2.shape[1]
    c3 = w3.shape[1]
    w1p = jnp.zeros((Dp, c1), jnp.float32).at[:cin].set(w1)
    wc = jnp.zeros((8, c1), jnp.float32).at[:3].set(w1[:3])
    return pl.pallas_call(
        functools.partial(_mlp_body, St=St, K=K),
        grid=(SB // St,),
        in_specs=[
            pl.BlockSpec((St * K, Dp), lambda s: (s, 0)),
            pl.BlockSpec((St, 8), lambda s: (s, 0)),
            pl.BlockSpec((Dp, c1), lambda s: (0, 0)),
            pl.BlockSpec((1, c1), lambda s: (0, 0)),
            pl.BlockSpec((8, c1), lambda s: (0, 0)),
            pl.BlockSpec((c1, c2), lambda s: (0, 0)),
            pl.BlockSpec((1, c2), lambda s: (0, 0)),
            pl.BlockSpec((c2, c3), lambda s: (0, 0)),
            pl.BlockSpec((1, c3), lambda s: (0, 0)),
        ],
        out_specs=pl.BlockSpec((St, c3), lambda s: (s, 0)),
        out_shape=jax.ShapeDtypeStruct((SB, c3), jnp.float32),
    )(G, cent, w1p, b1.reshape(1, c1), wc, w2, b2.reshape(1, c2),
      w3, b3.reshape(1, c3))


def _head_body(g_ref, w1_ref, b1_ref, w2_ref, b2_ref, w3_ref, b3_ref, o_ref,
               *, B, K):
    g = g_ref[...]
    h = jnp.maximum(jnp.dot(g, w1_ref[...], preferred_element_type=jnp.float32)
                    + b1_ref[...], 0.0)
    h = jnp.maximum(jnp.dot(h, w2_ref[...], preferred_element_type=jnp.float32)
                    + b2_ref[...], 0.0)
    h = jnp.maximum(jnp.dot(h, w3_ref[...], preferred_element_type=jnp.float32)
                    + b3_ref[...], 0.0)
    o_ref[...] = jnp.max(h.reshape(B, K, h.shape[-1]), axis=1)


def _head_mlp(G, layers):
    B, K, Dp = G.shape
    (w1, b1), (w2, b2), (w3, b3) = layers
    cin, c1 = w1.shape
    c2 = w2.shape[1]
    c3 = w3.shape[1]
    w1p = jnp.zeros((Dp, c1), jnp.float32).at[:cin].set(w1)
    return pl.pallas_call(
        functools.partial(_head_body, B=B, K=K),
        out_shape=jax.ShapeDtypeStruct((B, c3), jnp.float32),
    )(G.reshape(B * K, Dp), w1p, b1.reshape(1, c1), w2, b2.reshape(1, c2),
      w3, b3.reshape(1, c3))


def _sa_stage(xyz, feats, S, radius, K, layers, bq_tile, mlp_tile, gather_ch):
    B, N, _ = xyz.shape
    x, y, z = xyz[..., 0], xyz[..., 1], xyz[..., 2]
    cx, cy, cz = _fps(x, y, z, S)
    new_xyz = jnp.stack([cx, cy, cz], axis=-1)
    nxp = jnp.pad(new_xyz, ((0, 0), (0, 0), (0, 5)))
    xpt = jnp.pad(xyz, ((0, 0), (0, 0), (0, 5))).transpose(0, 2, 1)
    gidx = _ball_query(nxp, xpt, radius, K, bq_tile)

    C = 0 if feats is None else feats.shape[-1]
    Dp = _pad16(3 + C)
    table = jnp.zeros((B * N, Dp), jnp.float32)
    table = table.at[:, :3].set(xyz.reshape(B * N, 3))
    if feats is not None:
        table = table.at[:, 3:3 + C].set(feats.reshape(B * N, C))
    G = _sc_gather(table, gidx.reshape(-1), gather_ch)
    cent = nxp.reshape(B * S, 8)
    fo = _group_mlp(G, cent, layers, K, mlp_tile)
    return new_xyz, fo.reshape(B, S, -1)


def kernel(pointcloud, params):
    xyz = pointcloud[..., :3]
    B = xyz.shape[0]
    feats = None
    cfg = [
        (1024, 0.1, 32, 256, 128, 1024),
        (256, 0.2, 32, 256, 64, 512),
        (64, 0.4, 32, 64, 16, 512),
    ]
    for i, (S, radius, K, bq_tile, mlp_tile, gather_ch) in enumerate(cfg):
        xyz, feats = _sa_stage(xyz, feats, S, radius, K, params[i],
                               bq_tile, mlp_tile, gather_ch)
    Kf = xyz.shape[1]
    cin = 3 + feats.shape[-1]
    Dp = _pad16(cin)
    g = jnp.zeros((B, Kf, Dp), jnp.float32)
    g = g.at[:, :, :3].set(xyz)
    g = g.at[:, :, 3:cin].set(feats)
    return _head_mlp(g, params[3])


# speedup vs baseline: 70.7938x; 1.6877x over previous
import functools

import jax
import jax.numpy as jnp
from jax import lax
from jax.experimental import pallas as pl
from jax.experimental.pallas import tpu as pltpu
from jax.experimental.pallas import tpu_sc as plsc

_NC, _NS = 2, 16
_NW = _NC * _NS


def _pad16(c):
    return (c + 15) // 16 * 16


def _fps_body(x_ref, y_ref, z_ref, cx_ref, cy_ref, cz_ref, *, N, S):
    x = x_ref[...]
    y = y_ref[...]
    z = z_ref[...]
    B = x.shape[0]
    col = lax.broadcasted_iota(jnp.int32, (B, N), 1)
    row = lax.broadcasted_iota(jnp.int32, (B, N), 0)

    col_s = lax.broadcasted_iota(jnp.int32, (B, S), 1)
    row_s = lax.broadcasted_iota(jnp.int32, (B, S), 0)

    def body(i, state):
        dists, far, cxs, cys, czs = state
        sel = col_s == i
        cmask = col == far
        cx = jnp.sum(jnp.where(cmask, x, 0.0), axis=1, keepdims=True)
        cy = jnp.sum(jnp.where(cmask, y, 0.0), axis=1, keepdims=True)
        cz = jnp.sum(jnp.where(cmask, z, 0.0), axis=1, keepdims=True)
        cxs = jnp.where(sel, cx, cxs)
        cys = jnp.where(sel, cy, cys)
        czs = jnp.where(sel, cz, czs)
        d = ((x - cx) ** 2 + (y - cy) ** 2) + (z - cz) ** 2
        dists = jnp.minimum(dists, d)
        mx = jnp.max(dists, axis=1, keepdims=True)
        far = jnp.min(jnp.where(dists == mx, col, N), axis=1, keepdims=True)
        return dists, far, cxs, cys, czs

    zero2d = (row + col) * 0
    zs_f = ((row_s + col_s) * 0).astype(jnp.float32)
    init = (
        zero2d.astype(jnp.float32) + jnp.float32(1e10),
        jnp.min(zero2d, axis=1, keepdims=True),
        zs_f,
        zs_f,
        zs_f,
    )
    _, _, cxs, cys, czs = lax.fori_loop(0, S, body, init)
    cx_ref[...] = cxs
    cy_ref[...] = cys
    cz_ref[...] = czs


def _fps(x, y, z, S):
    B, N = x.shape
    out = (
        jax.ShapeDtypeStruct((B, S), jnp.float32),
        jax.ShapeDtypeStruct((B, S), jnp.float32),
        jax.ShapeDtypeStruct((B, S), jnp.float32),
    )
    return pl.pallas_call(
        functools.partial(_fps_body, N=N, S=S),
        out_shape=out,
    )(x, y, z)


def _bq_body(nxp_ref, xpt_ref, gidx_ref, *, N, K, r2, St):
    b = pl.program_id(0)
    nx = nxp_ref[0]
    xt = xpt_ref[0]
    dot = jnp.dot(nx, xt, preferred_element_type=jnp.float32)
    ns = (nx[:, 0:1] * nx[:, 0:1] + nx[:, 1:2] * nx[:, 1:2]) + nx[:, 2:3] * nx[:, 2:3]
    xs = (xt[0:1] * xt[0:1] + xt[1:2] * xt[1:2]) + xt[2:3] * xt[2:3]
    sq = ns + xs - 2.0 * dot
    maskf = jnp.where(sq <= r2, 1.0, 0.0)

    li = lax.broadcasted_iota(jnp.int32, (128, 8), 0)
    hi = lax.broadcasted_iota(jnp.int32, (128, 8), 1)
    pow2 = lax.bitcast_convert_type(((li % 16) + 127) << 23, jnp.float32)
    pmat = jnp.where(li // 16 == hi, pow2, 0.0)
    NH = N // 16
    words = jnp.concatenate(
        [jnp.dot(maskf[:, c * 128:(c + 1) * 128], pmat,
                 preferred_element_type=jnp.float32)
         for c in range(N // 128)], axis=1).astype(jnp.int32)

    hcol = lax.broadcasted_iota(jnp.int32, (St, NH), 1)
    col_k = lax.broadcasted_iota(jnp.int32, (St, K), 1)

    def extract(words):
        nz = words != 0
        widx = jnp.min(jnp.where(nz, hcol, NH), axis=1, keepdims=True)
        sel = hcol == widx
        wval = jnp.sum(jnp.where(sel, words, 0), axis=1, keepdims=True)
        lsb = wval & -wval
        bit = (lax.bitcast_convert_type(lsb.astype(jnp.float32), jnp.int32)
               >> 23) - 127
        j = widx * 16 + bit
        words = jnp.where(sel, wval & (wval - 1), words)
        return words, widx, j

    words, widx0, first = extract(words)
    out0 = jnp.where(col_k == 0, first, jnp.zeros((St, K), jnp.int32))

    def body(k, state):
        words, out = state
        words, widx, j = extract(words)
        val = jnp.where(widx == NH, first, j)
        out = jnp.where(col_k == k, val, out)
        return words, out

    _, out = lax.fori_loop(1, K, body, (words, out0))
    gidx_ref[0] = out + b * N


def _ball_query(nxp, xpt, radius, K, St):
    B, S, _ = nxp.shape
    N = xpt.shape[2]
    r2 = float(radius * radius)
    return pl.pallas_call(
        functools.partial(_bq_body, N=N, K=K, r2=r2, St=St),
        grid=(B, S // St),
        in_specs=[
            pl.BlockSpec((1, St, 8), lambda b, s: (b, s, 0)),
            pl.BlockSpec((1, 8, N), lambda b, s: (b, 0, 0)),
        ],
        out_specs=pl.BlockSpec((1, St, K), lambda b, s: (b, s, 0)),
        out_shape=jax.ShapeDtypeStruct((B, S, K), jnp.int32),
    )(nxp, xpt)


def _sc_gather(table, idx, CH):
    R, D = table.shape
    M = idx.shape[0]
    per_w = M // _NW
    assert per_w * _NW == M and per_w % CH == 0 and D % 16 == 0
    mesh = plsc.VectorSubcoreMesh(core_axis_name="c", subcore_axis_name="s",
                                  num_cores=_NC, num_subcores=_NS)

    @functools.partial(
        pl.kernel,
        mesh=mesh,
        out_type=jax.ShapeDtypeStruct((M, D), jnp.float32),
        scratch_types=[
            pltpu.VMEM((CH,), jnp.int32),
            pltpu.VMEM((CH, D), jnp.float32),
            pltpu.SemaphoreType.DMA,
        ],
        compiler_params=pltpu.CompilerParams(use_tc_tiling_on_sc=False),
    )
    def k(table_hbm, idx_hbm, out_hbm, idx_v, rows_v, sem):
        wid = lax.axis_index("s") * _NC + lax.axis_index("c")
        base = wid * per_w

        def step(i, carry):
            off = base + i * CH
            pltpu.sync_copy(idx_hbm.at[pl.ds(off, CH)], idx_v)
            pltpu.async_copy(table_hbm.at[idx_v], rows_v, sem).wait()
            pltpu.sync_copy(rows_v, out_hbm.at[pl.ds(off, CH)])
            return carry

        lax.fori_loop(0, per_w // CH, step, 0)

    return k(table, idx)


def _mlp_body(g_ref, c_ref, w1_ref, b1_ref, wc_ref, w2_ref, b2_ref, w3_ref,
              b3_ref, o_ref, *, St, K):
    g = g_ref[...]
    c = c_ref[...]
    cterm = jnp.dot(c, wc_ref[...], preferred_element_type=jnp.float32)
    h = jnp.dot(g, w1_ref[...], preferred_element_type=jnp.float32) + b1_ref[...]
    h = h.reshape(St, K, -1) - cterm[:, None, :]
    h = jnp.maximum(h, 0.0).reshape(St * K, h.shape[-1])
    h = jnp.maximum(jnp.dot(h, w2_ref[...], preferred_element_type=jnp.float32)
                    + b2_ref[...], 0.0)
    h = jnp.maximum(jnp.dot(h, w3_ref[...], preferred_element_type=jnp.float32)
                    + b3_ref[...], 0.0)
    o_ref[...] = jnp.max(h.reshape(St, K, h.shape[-1]), axis=1)


def _group_mlp(G, cent, layers, K, St):
    M, Dp = G.shape
    SB = M // K
    (w1, b1), (w2, b2), (w3, b3) = layers
    cin, c1 = w1.shape
    c2 = w2.shape[1]
    c3 = w3.shape[1]
    w1p = jnp.zeros((Dp, c1), jnp.float32).at[:cin].set(w1)
    wc = jnp.zeros((8, c1), jnp.float32).at[:3].set(w1[:3])
    return pl.pallas_call(
        functools.partial(_mlp_body, St=St, K=K),
        grid=(SB // St,),
        in_specs=[
            pl.BlockSpec((St * K, Dp), lambda s: (s, 0)),
            pl.BlockSpec((St, 8), lambda s: (s, 0)),
            pl.BlockSpec((Dp, c1), lambda s: (0, 0)),
            pl.BlockSpec((1, c1), lambda s: (0, 0)),
            pl.BlockSpec((8, c1), lambda s: (0, 0)),
            pl.BlockSpec((c1, c2), lambda s: (0, 0)),
            pl.BlockSpec((1, c2), lambda s: (0, 0)),
            pl.BlockSpec((c2, c3), lambda s: (0, 0)),
            pl.BlockSpec((1, c3), lambda s: (0, 0)),
        ],
        out_specs=pl.BlockSpec((St, c3), lambda s: (s, 0)),
        out_shape=jax.ShapeDtypeStruct((SB, c3), jnp.float32),
    )(G, cent, w1p, b1.reshape(1, c1), wc, w2, b2.reshape(1, c2),
      w3, b3.reshape(1, c3))


def _head_body(g_ref, w1_ref, b1_ref, w2_ref, b2_ref, w3_ref, b3_ref, o_ref,
               *, B, K):
    g = g_ref[...]
    h = jnp.maximum(jnp.dot(g, w1_ref[...], preferred_element_type=jnp.float32)
                    + b1_ref[...], 0.0)
    h = jnp.maximum(jnp.dot(h, w2_ref[...], preferred_element_type=jnp.float32)
                    + b2_ref[...], 0.0)
    h = jnp.maximum(jnp.dot(h, w3_ref[...], preferred_element_type=jnp.float32)
                    + b3_ref[...], 0.0)
    o_ref[...] = jnp.max(h.reshape(B, K, h.shape[-1]), axis=1)


def _head_mlp(G, layers):
    B, K, Dp = G.shape
    (w1, b1), (w2, b2), (w3, b3) = layers
    cin, c1 = w1.shape
    c2 = w2.shape[1]
    c3 = w3.shape[1]
    w1p = jnp.zeros((Dp, c1), jnp.float32).at[:cin].set(w1)
    return pl.pallas_call(
        functools.partial(_head_body, B=B, K=K),
        out_shape=jax.ShapeDtypeStruct((B, c3), jnp.float32),
    )(G.reshape(B * K, Dp), w1p, b1.reshape(1, c1), w2, b2.reshape(1, c2),
      w3, b3.reshape(1, c3))


def _sa_stage(xyz, feats, S, radius, K, layers, bq_tile, mlp_tile, gather_ch):
    B, N, _ = xyz.shape
    x, y, z = xyz[..., 0], xyz[..., 1], xyz[..., 2]
    cx, cy, cz = _fps(x, y, z, S)
    new_xyz = jnp.stack([cx, cy, cz], axis=-1)
    nxp = jnp.pad(new_xyz, ((0, 0), (0, 0), (0, 5)))
    xpt = jnp.pad(xyz, ((0, 0), (0, 0), (0, 5))).transpose(0, 2, 1)
    gidx = _ball_query(nxp, xpt, radius, K, bq_tile)

    C = 0 if feats is None else feats.shape[-1]
    Dp = _pad16(3 + C)
    table = jnp.zeros((B * N, Dp), jnp.float32)
    table = table.at[:, :3].set(xyz.reshape(B * N, 3))
    if feats is not None:
        table = table.at[:, 3:3 + C].set(feats.reshape(B * N, C))
    G = _sc_gather(table, gidx.reshape(-1), gather_ch)
    cent = nxp.reshape(B * S, 8)
    fo = _group_mlp(G, cent, layers, K, mlp_tile)
    return new_xyz, fo.reshape(B, S, -1)


def kernel(pointcloud, params):
    xyz = pointcloud[..., :3]
    B = xyz.shape[0]
    feats = None
    cfg = [
        (1024, 0.1, 32, 256, 128, 1024),
        (256, 0.2, 32, 256, 64, 512),
        (64, 0.4, 32, 64, 16, 512),
    ]
    for i, (S, radius, K, bq_tile, mlp_tile, gather_ch) in enumerate(cfg):
        xyz, feats = _sa_stage(xyz, feats, S, radius, K, params[i],
                               bq_tile, mlp_tile, gather_ch)
    Kf = xyz.shape[1]
    cin = 3 + feats.shape[-1]
    Dp = _pad16(cin)
    g = jnp.zeros((B, Kf, Dp), jnp.float32)
    g = g.at[:, :, :3].set(xyz)
    g = g.at[:, :, 3:cin].set(feats)
    return _head_mlp(g, params[3])


# speedup vs baseline: 72.7562x; 1.0277x over previous
import functools

import jax
import jax.numpy as jnp
from jax import lax
from jax.experimental import pallas as pl
from jax.experimental.pallas import tpu as pltpu
from jax.experimental.pallas import tpu_sc as plsc

_NC, _NS = 2, 16
_NW = _NC * _NS


def _pad16(c):
    return (c + 15) // 16 * 16


def _fps_body(x_ref, y_ref, z_ref, cx_ref, cy_ref, cz_ref, *, N, S):
    x = x_ref[...]
    y = y_ref[...]
    z = z_ref[...]
    B = x.shape[0]
    col = lax.broadcasted_iota(jnp.int32, (B, N), 1)
    row = lax.broadcasted_iota(jnp.int32, (B, N), 0)

    col_s = lax.broadcasted_iota(jnp.int32, (B, S), 1)
    row_s = lax.broadcasted_iota(jnp.int32, (B, S), 0)

    def body(i, state):
        dists, far, cxs, cys, czs = state
        sel = col_s == i
        cmask = col == far
        cx = jnp.sum(jnp.where(cmask, x, 0.0), axis=1, keepdims=True)
        cy = jnp.sum(jnp.where(cmask, y, 0.0), axis=1, keepdims=True)
        cz = jnp.sum(jnp.where(cmask, z, 0.0), axis=1, keepdims=True)
        cxs = jnp.where(sel, cx, cxs)
        cys = jnp.where(sel, cy, cys)
        czs = jnp.where(sel, cz, czs)
        d = ((x - cx) ** 2 + (y - cy) ** 2) + (z - cz) ** 2
        dists = jnp.minimum(dists, d)
        mx = jnp.max(dists, axis=1, keepdims=True)
        far = jnp.min(jnp.where(dists == mx, col, N), axis=1, keepdims=True)
        return dists, far, cxs, cys, czs

    zero2d = (row + col) * 0
    zs_f = ((row_s + col_s) * 0).astype(jnp.float32)
    init = (
        zero2d.astype(jnp.float32) + jnp.float32(1e10),
        jnp.min(zero2d, axis=1, keepdims=True),
        zs_f,
        zs_f,
        zs_f,
    )
    _, _, cxs, cys, czs = lax.fori_loop(0, S, body, init)
    cx_ref[...] = cxs
    cy_ref[...] = cys
    cz_ref[...] = czs


def _fps(x, y, z, S):
    B, N = x.shape
    out = (
        jax.ShapeDtypeStruct((B, S), jnp.float32),
        jax.ShapeDtypeStruct((B, S), jnp.float32),
        jax.ShapeDtypeStruct((B, S), jnp.float32),
    )
    return pl.pallas_call(
        functools.partial(_fps_body, N=N, S=S),
        out_shape=out,
    )(x, y, z)


def _bq_body(nxp_ref, xpt_ref, gidx_ref, *, N, K, r2, St):
    b = pl.program_id(0)
    nx = nxp_ref[0]
    xt = xpt_ref[0]
    dot = jnp.dot(nx, xt, preferred_element_type=jnp.float32)
    ns = (nx[:, 0:1] * nx[:, 0:1] + nx[:, 1:2] * nx[:, 1:2]) + nx[:, 2:3] * nx[:, 2:3]
    xs = (xt[0:1] * xt[0:1] + xt[1:2] * xt[1:2]) + xt[2:3] * xt[2:3]
    sq = ns + xs - 2.0 * dot
    maskf = jnp.where(sq <= r2, 1.0, 0.0)

    li = lax.broadcasted_iota(jnp.int32, (128, 8), 0)
    hi = lax.broadcasted_iota(jnp.int32, (128, 8), 1)
    pow2 = lax.bitcast_convert_type(((li % 16) + 127) << 23, jnp.float32)
    pmat = jnp.where(li // 16 == hi, pow2, 0.0)
    NH = N // 16
    words = jnp.concatenate(
        [jnp.dot(maskf[:, c * 128:(c + 1) * 128], pmat,
                 preferred_element_type=jnp.float32)
         for c in range(N // 128)], axis=1).astype(jnp.int32)

    hcol = lax.broadcasted_iota(jnp.int32, (St, NH), 1)
    col_k = lax.broadcasted_iota(jnp.int32, (St, K), 1)

    def extract(words):
        nz = words != 0
        widx = jnp.min(jnp.where(nz, hcol, NH), axis=1, keepdims=True)
        sel = hcol == widx
        wval = jnp.sum(jnp.where(sel, words, 0), axis=1, keepdims=True)
        lsb = wval & -wval
        bit = (lax.bitcast_convert_type(lsb.astype(jnp.float32), jnp.int32)
               >> 23) - 127
        j = widx * 16 + bit
        words = jnp.where(sel, wval & (wval - 1), words)
        return words, widx, j

    words, widx0, first = extract(words)
    out0 = jnp.where(col_k == 0, first, jnp.zeros((St, K), jnp.int32))

    def body(k, state):
        words, out = state
        words, widx, j = extract(words)
        val = jnp.where(widx == NH, first, j)
        out = jnp.where(col_k == k, val, out)
        return words, out

    _, out = lax.fori_loop(1, K, body, (words, out0))
    gidx_ref[0] = out + b * N


def _ball_query(nxp, xpt, radius, K, St):
    B, S, _ = nxp.shape
    N = xpt.shape[2]
    r2 = float(radius * radius)
    return pl.pallas_call(
        functools.partial(_bq_body, N=N, K=K, r2=r2, St=St),
        grid=(B, S // St),
        in_specs=[
            pl.BlockSpec((1, St, 8), lambda b, s: (b, s, 0)),
            pl.BlockSpec((1, 8, N), lambda b, s: (b, 0, 0)),
        ],
        out_specs=pl.BlockSpec((1, St, K), lambda b, s: (b, s, 0)),
        out_shape=jax.ShapeDtypeStruct((B, S, K), jnp.int32),
    )(nxp, xpt)


def _sc_gather(table, idx, CH):
    R, D = table.shape
    M = idx.shape[0]
    per_w = M // _NW
    assert per_w * _NW == M and per_w % CH == 0 and D % 16 == 0
    mesh = plsc.VectorSubcoreMesh(core_axis_name="c", subcore_axis_name="s",
                                  num_cores=_NC, num_subcores=_NS)

    @functools.partial(
        pl.kernel,
        mesh=mesh,
        out_type=jax.ShapeDtypeStruct((M, D), jnp.float32),
        scratch_types=[
            pltpu.VMEM((CH,), jnp.int32),
            pltpu.VMEM((CH, D), jnp.float32),
            pltpu.SemaphoreType.DMA,
        ],
        compiler_params=pltpu.CompilerParams(use_tc_tiling_on_sc=False),
    )
    def k(table_hbm, idx_hbm, out_hbm, idx_v, rows_v, sem):
        wid = lax.axis_index("s") * _NC + lax.axis_index("c")
        base = wid * per_w

        def step(i, carry):
            off = base + i * CH
            pltpu.sync_copy(idx_hbm.at[pl.ds(off, CH)], idx_v)
            pltpu.async_copy(table_hbm.at[idx_v], rows_v, sem).wait()
            pltpu.sync_copy(rows_v, out_hbm.at[pl.ds(off, CH)])
            return carry

        lax.fori_loop(0, per_w // CH, step, 0)

    return k(table, idx)


def _mlp_body(g_ref, c_ref, w1_ref, b1_ref, wc_ref, w2_ref, b2_ref, w3_ref,
              b3_ref, o_ref, *, St, K):
    g = g_ref[...]
    c = c_ref[...]
    cterm = jnp.dot(c, wc_ref[...], preferred_element_type=jnp.float32)
    h = jnp.dot(g, w1_ref[...], preferred_element_type=jnp.float32) + b1_ref[...]
    h = h.reshape(St, K, -1) - cterm[:, None, :]
    h = jnp.maximum(h, 0.0).reshape(St * K, h.shape[-1])
    h = jnp.maximum(jnp.dot(h, w2_ref[...], preferred_element_type=jnp.float32)
                    + b2_ref[...], 0.0)
    h = jnp.maximum(jnp.dot(h, w3_ref[...], preferred_element_type=jnp.float32)
                    + b3_ref[...], 0.0)
    o_ref[...] = jnp.max(h.reshape(St, K, h.shape[-1]), axis=1)


def _group_mlp(G, cent, layers, K, St):
    M, Dp = G.shape
    SB = M // K
    (w1, b1), (w2, b2), (w3, b3) = layers
    cin, c1 = w1.shape
    c2 = w2.shape[1]
    c3 = w3.shape[1]
    w1p = jnp.zeros((Dp, c1), jnp.float32).at[:cin].set(w1)
    wc = jnp.zeros((8, c1), jnp.float32).at[:3].set(w1[:3])
    return pl.pallas_call(
        functools.partial(_mlp_body, St=St, K=K),
        grid=(SB // St,),
        in_specs=[
            pl.BlockSpec((St * K, Dp), lambda s: (s, 0)),
            pl.BlockSpec((St, 8), lambda s: (s, 0)),
            pl.BlockSpec((Dp, c1), lambda s: (0, 0)),
            pl.BlockSpec((1, c1), lambda s: (0, 0)),
            pl.BlockSpec((8, c1), lambda s: (0, 0)),
            pl.BlockSpec((c1, c2), lambda s: (0, 0)),
            pl.BlockSpec((1, c2), lambda s: (0, 0)),
            pl.BlockSpec((c2, c3), lambda s: (0, 0)),
            pl.BlockSpec((1, c3), lambda s: (0, 0)),
        ],
        out_specs=pl.BlockSpec((St, c3), lambda s: (s, 0)),
        out_shape=jax.ShapeDtypeStruct((SB, c3), jnp.float32),
    )(G, cent, w1p, b1.reshape(1, c1), wc, w2, b2.reshape(1, c2),
      w3, b3.reshape(1, c3))


def _mlpg_body(gidx_ref, t_ref, c_ref, w1_ref, b1_ref, wc_ref, w2_ref, b2_ref,
               w3_ref, b3_ref, o_ref, *, St, K, N2):
    b = pl.program_id(0)
    loc = gidx_ref[0] - b * N2
    ncol = lax.broadcasted_iota(jnp.int32, (St, K, N2), 2)
    onehot = jnp.where(ncol == loc[:, :, None], 1.0, 0.0).reshape(St * K, N2)
    g = jnp.dot(onehot, t_ref[0], preferred_element_type=jnp.float32)
    c = c_ref[...]
    cterm = jnp.dot(c, wc_ref[...], preferred_element_type=jnp.float32)
    h = jnp.dot(g, w1_ref[...], preferred_element_type=jnp.float32) + b1_ref[...]
    h = h.reshape(St, K, -1) - cterm[:, None, :]
    h = jnp.maximum(h, 0.0).reshape(St * K, h.shape[-1])
    h = jnp.maximum(jnp.dot(h, w2_ref[...], preferred_element_type=jnp.float32)
                    + b2_ref[...], 0.0)
    h = jnp.maximum(jnp.dot(h, w3_ref[...], preferred_element_type=jnp.float32)
                    + b3_ref[...], 0.0)
    o_ref[...] = jnp.max(h.reshape(St, K, h.shape[-1]), axis=1)


def _group_mlp_onehot(gidx, table, cent, layers, K, St):
    B, S, _ = gidx.shape
    N2, Dp = table.shape[1], table.shape[2]
    (w1, b1), (w2, b2), (w3, b3) = layers
    cin, c1 = w1.shape
    c2 = w2.shape[1]
    c3 = w3.shape[1]
    w1p = jnp.zeros((Dp, c1), jnp.float32).at[:cin].set(w1)
    wc = jnp.zeros((8, c1), jnp.float32).at[:3].set(w1[:3])
    SpB = S // St
    return pl.pallas_call(
        functools.partial(_mlpg_body, St=St, K=K, N2=N2),
        grid=(B, SpB),
        in_specs=[
            pl.BlockSpec((1, St, K), lambda b, t: (b, t, 0)),
            pl.BlockSpec((1, N2, Dp), lambda b, t: (b, 0, 0)),
            pl.BlockSpec((St, 8), lambda b, t: (b * SpB + t, 0)),
            pl.BlockSpec((Dp, c1), lambda b, t: (0, 0)),
            pl.BlockSpec((1, c1), lambda b, t: (0, 0)),
            pl.BlockSpec((8, c1), lambda b, t: (0, 0)),
            pl.BlockSpec((c1, c2), lambda b, t: (0, 0)),
            pl.BlockSpec((1, c2), lambda b, t: (0, 0)),
            pl.BlockSpec((c2, c3), lambda b, t: (0, 0)),
            pl.BlockSpec((1, c3), lambda b, t: (0, 0)),
        ],
        out_specs=pl.BlockSpec((St, c3), lambda b, t: (b * SpB + t, 0)),
        out_shape=jax.ShapeDtypeStruct((B * S, c3), jnp.float32),
    )(gidx, table, cent, w1p, b1.reshape(1, c1), wc, w2, b2.reshape(1, c2),
      w3, b3.reshape(1, c3))


def _head_body(g_ref, w1_ref, b1_ref, w2_ref, b2_ref, w3_ref, b3_ref, o_ref,
               *, B, K):
    g = g_ref[...]
    h = jnp.maximum(jnp.dot(g, w1_ref[...], preferred_element_type=jnp.float32)
                    + b1_ref[...], 0.0)
    h = jnp.maximum(jnp.dot(h, w2_ref[...], preferred_element_type=jnp.float32)
                    + b2_ref[...], 0.0)
    h = jnp.maximum(jnp.dot(h, w3_ref[...], preferred_element_type=jnp.float32)
                    + b3_ref[...], 0.0)
    o_ref[...] = jnp.max(h.reshape(B, K, h.shape[-1]), axis=1)


def _head_mlp(G, layers):
    B, K, Dp = G.shape
    (w1, b1), (w2, b2), (w3, b3) = layers
    cin, c1 = w1.shape
    c2 = w2.shape[1]
    c3 = w3.shape[1]
    w1p = jnp.zeros((Dp, c1), jnp.float32).at[:cin].set(w1)
    return pl.pallas_call(
        functools.partial(_head_body, B=B, K=K),
        out_shape=jax.ShapeDtypeStruct((B, c3), jnp.float32),
    )(G.reshape(B * K, Dp), w1p, b1.reshape(1, c1), w2, b2.reshape(1, c2),
      w3, b3.reshape(1, c3))


def _sa_stage(xyz, feats, S, radius, K, layers, bq_tile, mlp_tile, gather_ch):
    B, N, _ = xyz.shape
    x, y, z = xyz[..., 0], xyz[..., 1], xyz[..., 2]
    cx, cy, cz = _fps(x, y, z, S)
    new_xyz = jnp.stack([cx, cy, cz], axis=-1)
    nxp = jnp.pad(new_xyz, ((0, 0), (0, 0), (0, 5)))
    xpt = jnp.pad(xyz, ((0, 0), (0, 0), (0, 5))).transpose(0, 2, 1)
    gidx = _ball_query(nxp, xpt, radius, K, bq_tile)

    C = 0 if feats is None else feats.shape[-1]
    Dp = _pad16(3 + C)
    table = jnp.zeros((B * N, Dp), jnp.float32)
    table = table.at[:, :3].set(xyz.reshape(B * N, 3))
    if feats is not None:
        table = table.at[:, 3:3 + C].set(feats.reshape(B * N, C))
    cent = nxp.reshape(B * S, 8)
    if N <= 256:
        fo = _group_mlp_onehot(gidx, table.reshape(B, N, Dp), cent, layers,
                               K, mlp_tile)
    else:
        G = _sc_gather(table, gidx.reshape(-1), gather_ch)
        fo = _group_mlp(G, cent, layers, K, mlp_tile)
    return new_xyz, fo.reshape(B, S, -1)


def kernel(pointcloud, params):
    xyz = pointcloud[..., :3]
    B = xyz.shape[0]
    feats = None
    cfg = [
        (1024, 0.1, 32, 256, 128, 1024),
        (256, 0.2, 32, 256, 64, 512),
        (64, 0.4, 32, 64, 16, 512),
    ]
    for i, (S, radius, K, bq_tile, mlp_tile, gather_ch) in enumerate(cfg):
        xyz, feats = _sa_stage(xyz, feats, S, radius, K, params[i],
                               bq_tile, mlp_tile, gather_ch)
    Kf = xyz.shape[1]
    cin = 3 + feats.shape[-1]
    Dp = _pad16(cin)
    g = jnp.zeros((B, Kf, Dp), jnp.float32)
    g = g.at[:, :, :3].set(xyz)
    g = g.at[:, :, 3:cin].set(feats)
    return _head_mlp(g, params[3])
